# baseline (speedup 1.0000x reference)
import dataclasses
import functools

import jax
import jax.numpy as jnp
from jax import lax
from jax.experimental import pallas as pl
from jax.experimental.pallas import tpu as pltpu
from jax.experimental.pallas import tpu_sc as plsc

_ENCODER_STRIDE = 320
_L = 16


@functools.partial(jax.jit, static_argnames=("W",))
def _sc_vq_loss_partials(student, codes_flat, codebook, nframes, W=128):
    B, D, T = student.shape
    NT = 32
    per_tile = (B * T) // NT
    tiles_per_b = NT // B
    n_blk = per_tile // W
    mesh = plsc.VectorSubcoreMesh(core_axis_name="c", subcore_axis_name="s")
    cp = pltpu.CompilerParams()
    if "needs_layout_passes" in pltpu.CompilerParams.__dataclass_fields__:
        cp = dataclasses.replace(cp, needs_layout_passes=False)

    @functools.partial(
        pl.kernel,
        compiler_params=cp,
        out_type=jax.ShapeDtypeStruct((NT, 2, _L), jnp.float32),
        mesh=mesh,
        scratch_types=[
            pltpu.VMEM((W,), jnp.int32),
            pltpu.VMEM((W, D), jnp.float32),
            pltpu.VMEM((D, W), jnp.float32),
            pltpu.VMEM((B,), jnp.int32),
            pltpu.VMEM((2, _L), jnp.float32),
        ],
    )
    def k(st_hbm, codes_hbm, cb_hbm, nf_hbm, out_hbm,
          idx_v, crows_v, sblk_v, nf_v, acc_v):
        cid = lax.axis_index("c")
        sid = lax.axis_index("s")
        wid = sid * 2 + cid
        b = wid // tiles_per_b
        t_base = (wid % tiles_per_b) * per_tile
        flat_base = wid * per_tile

        pltpu.sync_copy(nf_hbm, nf_v)
        nf_b = plsc.load_gather(nf_v, [jnp.full((_L,), b, jnp.int32)])
        acc_v[0, :] = jnp.zeros((_L,), jnp.float32)
        acc_v[1, :] = jnp.zeros((_L,), jnp.float32)

        @pl.loop(0, n_blk)
        def _blk(blk):
            t0 = t_base + blk * W
            pltpu.sync_copy(codes_hbm.at[pl.ds(flat_base + blk * W, W)], idx_v)
            pltpu.sync_copy(cb_hbm.at[idx_v], crows_v)
            pltpu.sync_copy(st_hbm.at[b, :, pl.ds(t0, W)], sblk_v)

            loss_acc = acc_v[0, :]
            cnt_acc = acc_v[1, :]
            for j in range(W // _L):
                t_loc = j * _L + lax.iota(jnp.int32, _L)

                def d_body(d, acc, t_loc=t_loc, j=j):
                    s16 = sblk_v[d, pl.ds(j * _L, _L)]
                    c16 = plsc.load_gather(
                        crows_v, [t_loc, jnp.full((_L,), d, jnp.int32)])
                    diff = s16 - c16
                    return acc + diff * diff

                sq = lax.fori_loop(0, D, d_body,
                                   jnp.zeros((_L,), jnp.float32), unroll=4)
                m = jnp.where(t0 + t_loc < nf_b, 1.0, 0.0).astype(jnp.float32)
                loss_acc = loss_acc + m * sq
                cnt_acc = cnt_acc + m
            acc_v[0, :] = loss_acc
            acc_v[1, :] = cnt_acc

        pltpu.sync_copy(acc_v, out_hbm.at[wid])

    return k(student, codes_flat, codebook, nframes)


def kernel(student_features, teacher_codes, codebook, lengths):
    if teacher_codes.ndim == 3:
        teacher_codes = teacher_codes[0]
    B, D, T = student_features.shape
    codes_flat = teacher_codes.reshape(-1).astype(jnp.int32)
    nframes = jnp.minimum(
        (lengths // _ENCODER_STRIDE).astype(jnp.int32), T).astype(jnp.int32)
    out = _sc_vq_loss_partials(
        student_features.astype(jnp.float32),
        codes_flat, codebook.astype(jnp.float32), nframes)
    loss_sum = out[:, 0, :].sum()
    cnt = out[:, 1, :].sum()
    return (loss_sum / D) / (cnt + 1e-8)


# speedup vs baseline: 1.0001x; 1.0001x over previous
import dataclasses
import functools

import jax
import jax.numpy as jnp
from jax import lax
from jax.experimental import pallas as pl
from jax.experimental.pallas import tpu as pltpu
from jax.experimental.pallas import tpu_sc as plsc

_ENCODER_STRIDE = 320
_L = 16


@functools.partial(jax.jit, static_argnames=("W",))
def _sc_vq_loss_partials(student, codes_flat, codebook, nframes, W=128):
    B, D, T = student.shape
    NT = 32
    per_tile = (B * T) // NT
    tiles_per_b = NT // B
    n_blk = per_tile // W
    mesh = plsc.VectorSubcoreMesh(core_axis_name="c", subcore_axis_name="s")
    cp = pltpu.CompilerParams()
    if "needs_layout_passes" in pltpu.CompilerParams.__dataclass_fields__:
        cp = dataclasses.replace(cp, needs_layout_passes=False)

    @functools.partial(
        pl.kernel,
        compiler_params=cp,
        out_type=jax.ShapeDtypeStruct((NT, 2, _L), jnp.float32),
        mesh=mesh,
        scratch_types=[
            pltpu.VMEM((W,), jnp.int32),
            pltpu.VMEM((W, D + 1), jnp.float32),
            pltpu.VMEM((D, W), jnp.float32),
            pltpu.VMEM((B,), jnp.int32),
            pltpu.VMEM((2, _L), jnp.float32),
        ],
    )
    def k(st_hbm, codes_hbm, cb_hbm, nf_hbm, out_hbm,
          idx_v, crows_v, sblk_v, nf_v, acc_v):
        cid = lax.axis_index("c")
        sid = lax.axis_index("s")
        wid = sid * 2 + cid
        b = wid // tiles_per_b
        t_base = (wid % tiles_per_b) * per_tile
        flat_base = wid * per_tile

        pltpu.sync_copy(nf_hbm, nf_v)
        nf_b = plsc.load_gather(nf_v, [jnp.full((_L,), b, jnp.int32)])
        acc_v[0, :] = jnp.zeros((_L,), jnp.float32)
        acc_v[1, :] = jnp.zeros((_L,), jnp.float32)

        @pl.loop(0, n_blk)
        def _blk(blk):
            t0 = t_base + blk * W
            pltpu.sync_copy(codes_hbm.at[pl.ds(flat_base + blk * W, W)], idx_v)
            pltpu.sync_copy(cb_hbm.at[idx_v], crows_v.at[:, pl.ds(0, D)])
            pltpu.sync_copy(st_hbm.at[b, :, pl.ds(t0, W)], sblk_v)

            loss_acc = acc_v[0, :]
            cnt_acc = acc_v[1, :]
            for j in range(W // _L):
                t_loc = j * _L + lax.iota(jnp.int32, _L)

                def d_body(d, acc, t_loc=t_loc, j=j):
                    s16 = sblk_v[d, pl.ds(j * _L, _L)]
                    c16 = plsc.load_gather(
                        crows_v, [t_loc, jnp.full((_L,), d, jnp.int32)])
                    diff = s16 - c16
                    return acc + diff * diff

                sq = lax.fori_loop(0, D, d_body,
                                   jnp.zeros((_L,), jnp.float32), unroll=4)
                m = jnp.where(t0 + t_loc < nf_b, 1.0, 0.0).astype(jnp.float32)
                loss_acc = loss_acc + m * sq
                cnt_acc = cnt_acc + m
            acc_v[0, :] = loss_acc
            acc_v[1, :] = cnt_acc

        pltpu.sync_copy(acc_v, out_hbm.at[wid])

    return k(student, codes_flat, codebook, nframes)


def kernel(student_features, teacher_codes, codebook, lengths):
    if teacher_codes.ndim == 3:
        teacher_codes = teacher_codes[0]
    B, D, T = student_features.shape
    codes_flat = teacher_codes.reshape(-1).astype(jnp.int32)
    nframes = jnp.minimum(
        (lengths // _ENCODER_STRIDE).astype(jnp.int32), T).astype(jnp.int32)
    out = _sc_vq_loss_partials(
        student_features.astype(jnp.float32),
        codes_flat, codebook.astype(jnp.float32), nframes)
    loss_sum = out[:, 0, :].sum()
    cnt = out[:, 1, :].sum()
    return (loss_sum / D) / (cnt + 1e-8)


# speedup vs baseline: 1.5377x; 1.5376x over previous
import dataclasses
import functools

import jax
import jax.numpy as jnp
from jax import lax
from jax.experimental import pallas as pl
from jax.experimental.pallas import tpu as pltpu
from jax.experimental.pallas import tpu_sc as plsc

_ENCODER_STRIDE = 320
_L = 16


def _tc_transpose(student):
    B, D, T = student.shape
    TT = 512

    def body(x_ref, o_ref):
        o_ref[0] = jnp.swapaxes(x_ref[0], 0, 1)

    return pl.pallas_call(
        body,
        grid=(B, T // TT),
        in_specs=[pl.BlockSpec((1, D, TT), lambda b, t: (b, 0, t))],
        out_specs=pl.BlockSpec((1, TT, D), lambda b, t: (b, t, 0)),
        out_shape=jax.ShapeDtypeStruct((B, T, D), jnp.float32),
    )(student)


@functools.partial(jax.jit, static_argnames=("W",))
def _sc_vq_loss_partials(student_t, codes_flat, codebook, nframes, W=128):
    B, T, D = student_t.shape
    NT = 32
    per_tile = (B * T) // NT
    tiles_per_b = NT // B
    n_blk = per_tile // W
    mesh = plsc.VectorSubcoreMesh(core_axis_name="c", subcore_axis_name="s")
    cp = pltpu.CompilerParams()
    if "needs_layout_passes" in pltpu.CompilerParams.__dataclass_fields__:
        cp = dataclasses.replace(cp, needs_layout_passes=False)

    @functools.partial(
        pl.kernel,
        compiler_params=cp,
        out_type=jax.ShapeDtypeStruct((NT, 2, _L), jnp.float32),
        mesh=mesh,
        scratch_types=[
            pltpu.VMEM((per_tile,), jnp.int32),
            pltpu.VMEM((W, D), jnp.float32),
            pltpu.VMEM((W, D), jnp.float32),
            pltpu.VMEM((B,), jnp.int32),
            pltpu.VMEM((2, _L), jnp.float32),
        ],
    )
    def k(st_hbm, codes_hbm, cb_hbm, nf_hbm, out_hbm,
          idx_v, crows_v, sblk_v, nf_v, acc_v):
        cid = lax.axis_index("c")
        sid = lax.axis_index("s")
        wid = sid * 2 + cid
        b = wid // tiles_per_b
        t_base = (wid % tiles_per_b) * per_tile
        flat_base = wid * per_tile

        pltpu.sync_copy(nf_hbm, nf_v)
        pltpu.sync_copy(codes_hbm.at[pl.ds(flat_base, per_tile)], idx_v)
        nf_b = plsc.load_gather(nf_v, [jnp.full((_L,), b, jnp.int32)])
        acc_v[0, :] = jnp.zeros((_L,), jnp.float32)
        acc_v[1, :] = jnp.zeros((_L,), jnp.float32)

        @pl.loop(0, n_blk)
        def _blk(blk):
            t0 = t_base + blk * W
            pltpu.sync_copy(cb_hbm.at[idx_v.at[pl.ds(blk * W, W)]], crows_v)
            pltpu.sync_copy(st_hbm.at[b, pl.ds(t0, W), :], sblk_v)

            def t_body(tl, carry):
                tot, cnt = carry
                acc = jnp.zeros((_L,), jnp.float32)
                for i in range(D // _L):
                    s16 = sblk_v[tl, pl.ds(i * _L, _L)]
                    c16 = crows_v[tl, pl.ds(i * _L, _L)]
                    diff = s16 - c16
                    acc = acc + diff * diff
                m = jnp.where(t0 + tl < nf_b, 1.0, 0.0).astype(jnp.float32)
                return tot + m * acc, cnt + m

            tot, cnt = lax.fori_loop(
                0, W, t_body, (acc_v[0, :], acc_v[1, :]))
            acc_v[0, :] = tot
            acc_v[1, :] = cnt

        pltpu.sync_copy(acc_v, out_hbm.at[wid])

    return k(student_t, codes_flat, codebook, nframes)


def kernel(student_features, teacher_codes, codebook, lengths):
    if teacher_codes.ndim == 3:
        teacher_codes = teacher_codes[0]
    B, D, T = student_features.shape
    codes_flat = teacher_codes.reshape(-1).astype(jnp.int32)
    nframes = jnp.minimum(
        (lengths // _ENCODER_STRIDE).astype(jnp.int32), T).astype(jnp.int32)
    student_t = _tc_transpose(student_features.astype(jnp.float32))
    out = _sc_vq_loss_partials(
        student_t, codes_flat, codebook.astype(jnp.float32), nframes)
    loss_sum = out[:, 0, :].sum()
    cnt = out[:, 1, :].sum() / _L
    return (loss_sum / D) / (cnt + 1e-8)


# speedup vs baseline: 1.8327x; 1.1918x over previous
import dataclasses
import functools

import jax
import jax.numpy as jnp
from jax import lax
from jax.experimental import pallas as pl
from jax.experimental.pallas import tpu as pltpu
from jax.experimental.pallas import tpu_sc as plsc

_ENCODER_STRIDE = 320
_L = 16


def _tc_transpose(student):
    B, D, T = student.shape
    TT = 512

    def body(x_ref, o_ref):
        o_ref[0] = jnp.swapaxes(x_ref[0], 0, 1)

    return pl.pallas_call(
        body,
        grid=(B, T // TT),
        in_specs=[pl.BlockSpec((1, D, TT), lambda b, t: (b, 0, t))],
        out_specs=pl.BlockSpec((1, TT, D), lambda b, t: (b, t, 0)),
        out_shape=jax.ShapeDtypeStruct((B, T, D), jnp.float32),
    )(student)


@functools.partial(jax.jit, static_argnames=("W",))
def _sc_vq_loss_partials(student_t, codes_flat, codebook, nframes, W=64):
    B, T, D = student_t.shape
    NT = 32
    per_tile = (B * T) // NT
    tiles_per_b = NT // B
    n_blk = per_tile // W
    assert n_blk % 2 == 0
    mesh = plsc.VectorSubcoreMesh(core_axis_name="c", subcore_axis_name="s")
    cp = pltpu.CompilerParams()
    if "needs_layout_passes" in pltpu.CompilerParams.__dataclass_fields__:
        cp = dataclasses.replace(cp, needs_layout_passes=False)

    @functools.partial(
        pl.kernel,
        compiler_params=cp,
        out_type=jax.ShapeDtypeStruct((NT, 2, _L), jnp.float32),
        mesh=mesh,
        scratch_types=[
            pltpu.VMEM((per_tile,), jnp.int32),
            pltpu.VMEM((2, W, D), jnp.float32),
            pltpu.VMEM((2, W, D), jnp.float32),
            pltpu.VMEM((B,), jnp.int32),
            pltpu.VMEM((2, _L), jnp.float32),
            pltpu.SemaphoreType.DMA,
            pltpu.SemaphoreType.DMA,
            pltpu.SemaphoreType.DMA,
            pltpu.SemaphoreType.DMA,
        ],
    )
    def k(st_hbm, codes_hbm, cb_hbm, nf_hbm, out_hbm,
          idx_v, crows_v, sblk_v, nf_v, acc_v,
          sem_c0, sem_s0, sem_c1, sem_s1):
        cid = lax.axis_index("c")
        sid = lax.axis_index("s")
        wid = sid * 2 + cid
        b = wid // tiles_per_b
        t_base = (wid % tiles_per_b) * per_tile
        flat_base = wid * per_tile
        sems = ((sem_c0, sem_s0), (sem_c1, sem_s1))

        pltpu.sync_copy(nf_hbm, nf_v)
        pltpu.sync_copy(codes_hbm.at[pl.ds(flat_base, per_tile)], idx_v)
        nf_b = plsc.load_gather(nf_v, [jnp.full((_L,), b, jnp.int32)])
        acc_v[0, :] = jnp.zeros((_L,), jnp.float32)
        acc_v[1, :] = jnp.zeros((_L,), jnp.float32)

        def copies(blk, buf):
            return (
                pltpu.make_async_copy(
                    cb_hbm.at[idx_v.at[pl.ds(blk * W, W)]],
                    crows_v.at[buf], sems[buf][0]),
                pltpu.make_async_copy(
                    st_hbm.at[b, pl.ds(t_base + blk * W, W), :],
                    sblk_v.at[buf], sems[buf][1]),
            )

        def start_blk(blk, buf):
            for cp_ in copies(blk, buf):
                cp_.start()

        def wait_blk(blk, buf):
            for cp_ in copies(blk, buf):
                cp_.wait()

        def compute(blk, buf):
            t0 = t_base + blk * W

            def t_body(tl, carry):
                tot, cnt = carry
                acc = jnp.zeros((_L,), jnp.float32)
                for i in range(D // _L):
                    s16 = sblk_v[buf, tl, pl.ds(i * _L, _L)]
                    c16 = crows_v[buf, tl, pl.ds(i * _L, _L)]
                    diff = s16 - c16
                    acc = acc + diff * diff
                m = jnp.where(t0 + tl < nf_b, 1.0, 0.0).astype(jnp.float32)
                return tot + m * acc, cnt + m

            tot, cnt = lax.fori_loop(0, W, t_body, (acc_v[0, :], acc_v[1, :]))
            acc_v[0, :] = tot
            acc_v[1, :] = cnt

        start_blk(0, 0)

        @pl.loop(0, n_blk, step=2)
        def _blk(blk):
            start_blk(blk + 1, 1)
            wait_blk(blk, 0)
            compute(blk, 0)

            @pl.when(blk + 2 < n_blk)
            def _():
                start_blk(blk + 2, 0)

            wait_blk(blk + 1, 1)
            compute(blk + 1, 1)

        pltpu.sync_copy(acc_v, out_hbm.at[wid])

    return k(student_t, codes_flat, codebook, nframes)


def kernel(student_features, teacher_codes, codebook, lengths):
    if teacher_codes.ndim == 3:
        teacher_codes = teacher_codes[0]
    B, D, T = student_features.shape
    codes_flat = teacher_codes.reshape(-1).astype(jnp.int32)
    nframes = jnp.minimum(
        (lengths // _ENCODER_STRIDE).astype(jnp.int32), T).astype(jnp.int32)
    student_t = _tc_transpose(student_features.astype(jnp.float32))
    out = _sc_vq_loss_partials(
        student_t, codes_flat, codebook.astype(jnp.float32), nframes)
    loss_sum = out[:, 0, :].sum()
    cnt = out[:, 1, :].sum() / _L
    return (loss_sum / D) / (cnt + 1e-8)


# speedup vs baseline: 2.3552x; 1.2851x over previous
import dataclasses
import functools

import jax
import jax.numpy as jnp
from jax import lax
from jax.experimental import pallas as pl
from jax.experimental.pallas import tpu as pltpu
from jax.experimental.pallas import tpu_sc as plsc

_ENCODER_STRIDE = 320
_L = 16


def _tc_transpose(student):
    B, D, T = student.shape
    TT = 2048

    def body(x_ref, o_ref):
        o_ref[0] = jnp.swapaxes(x_ref[0], 0, 1)

    return pl.pallas_call(
        body,
        grid=(B, T // TT),
        in_specs=[pl.BlockSpec((1, D, TT), lambda b, t: (b, 0, t))],
        out_specs=pl.BlockSpec((1, TT, D), lambda b, t: (b, t, 0)),
        out_shape=jax.ShapeDtypeStruct((B, T, D), jnp.float32),
    )(student)


@functools.partial(jax.jit, static_argnames=("W",))
def _sc_vq_loss_partials(student_t, codes_flat, codebook, nframes, W=64):
    B, T, D = student_t.shape
    NT = 32
    per_tile = (B * T) // NT
    tiles_per_b = NT // B
    n_blk = per_tile // W
    assert n_blk % 2 == 0
    mesh = plsc.VectorSubcoreMesh(core_axis_name="c", subcore_axis_name="s")
    cp = pltpu.CompilerParams()
    if "needs_layout_passes" in pltpu.CompilerParams.__dataclass_fields__:
        cp = dataclasses.replace(cp, needs_layout_passes=False)

    @functools.partial(
        pl.kernel,
        compiler_params=cp,
        out_type=jax.ShapeDtypeStruct((NT, 2, _L), jnp.float32),
        mesh=mesh,
        scratch_types=[
            pltpu.VMEM((per_tile,), jnp.int32),
            pltpu.VMEM((2, W, D), jnp.float32),
            pltpu.VMEM((2, W, D), jnp.float32),
            pltpu.VMEM((B,), jnp.int32),
            pltpu.VMEM((2, _L), jnp.float32),
            pltpu.SemaphoreType.DMA,
            pltpu.SemaphoreType.DMA,
            pltpu.SemaphoreType.DMA,
            pltpu.SemaphoreType.DMA,
        ],
    )
    def k(st_hbm, codes_hbm, cb_hbm, nf_hbm, out_hbm,
          idx_v, crows_v, sblk_v, nf_v, acc_v,
          sem_c0, sem_s0, sem_c1, sem_s1):
        cid = lax.axis_index("c")
        sid = lax.axis_index("s")
        wid = sid * 2 + cid
        b = wid // tiles_per_b
        t_base = (wid % tiles_per_b) * per_tile
        flat_base = wid * per_tile
        sems = ((sem_c0, sem_s0), (sem_c1, sem_s1))

        pltpu.sync_copy(nf_hbm, nf_v)
        pltpu.sync_copy(codes_hbm.at[pl.ds(flat_base, per_tile)], idx_v)
        nf_b = plsc.load_gather(nf_v, [jnp.full((_L,), b, jnp.int32)])
        acc_v[0, :] = jnp.zeros((_L,), jnp.float32)
        acc_v[1, :] = jnp.zeros((_L,), jnp.float32)

        def copies(blk, buf):
            return (
                pltpu.make_async_copy(
                    cb_hbm.at[idx_v.at[pl.ds(blk * W, W)]],
                    crows_v.at[buf], sems[buf][0]),
                pltpu.make_async_copy(
                    st_hbm.at[b, pl.ds(t_base + blk * W, W), :],
                    sblk_v.at[buf], sems[buf][1]),
            )

        def start_blk(blk, buf):
            for cp_ in copies(blk, buf):
                cp_.start()

        def wait_blk(blk, buf):
            for cp_ in copies(blk, buf):
                cp_.wait()

        def compute(blk, buf):
            t0 = t_base + blk * W

            def t_body(tl, carry):
                tot, cnt = carry
                acc = jnp.zeros((_L,), jnp.float32)
                for i in range(D // _L):
                    s16 = sblk_v[buf, tl, pl.ds(i * _L, _L)]
                    c16 = crows_v[buf, tl, pl.ds(i * _L, _L)]
                    diff = s16 - c16
                    acc = acc + diff * diff
                m = jnp.where(t0 + tl < nf_b, 1.0, 0.0).astype(jnp.float32)
                return tot + m * acc, cnt + m

            tot, cnt = lax.fori_loop(0, W, t_body, (acc_v[0, :], acc_v[1, :]))
            acc_v[0, :] = tot
            acc_v[1, :] = cnt

        start_blk(0, 0)

        @pl.loop(0, n_blk, step=2)
        def _blk(blk):
            start_blk(blk + 1, 1)
            wait_blk(blk, 0)
            compute(blk, 0)

            @pl.when(blk + 2 < n_blk)
            def _():
                start_blk(blk + 2, 0)

            wait_blk(blk + 1, 1)
            compute(blk + 1, 1)

        pltpu.sync_copy(acc_v, out_hbm.at[wid])

    return k(student_t, codes_flat, codebook, nframes)


def kernel(student_features, teacher_codes, codebook, lengths):
    if teacher_codes.ndim == 3:
        teacher_codes = teacher_codes[0]
    B, D, T = student_features.shape
    codes_flat = teacher_codes.reshape(-1).astype(jnp.int32)
    nframes = jnp.minimum(
        (lengths // _ENCODER_STRIDE).astype(jnp.int32), T).astype(jnp.int32)
    student_t = _tc_transpose(student_features.astype(jnp.float32))
    out = _sc_vq_loss_partials(
        student_t, codes_flat, codebook.astype(jnp.float32), nframes)
    loss_sum = out[:, 0, :].sum()
    cnt = out[:, 1, :].sum() / _L
    return (loss_sum / D) / (cnt + 1e-8)
